# baseline (speedup 1.0000x reference)
import jax
import jax.numpy as jnp
from jax.experimental import pallas as pl
from jax.experimental.pallas import tpu as pltpu
from jax.experimental.pallas import tpu_sc as plsc

D_MODEL = 512
SQUARES = 64
BB = 64
GATHER_WINDOW = 128


def _sc_gather(pairsum_i32, idx):
    n = idx.shape[1]
    d_half = D_MODEL // 2
    mesh = plsc.VectorSubcoreMesh(core_axis_name="c", subcore_axis_name="s")

    @pl.kernel(out_type=jax.ShapeDtypeStruct((n, d_half), jnp.int32),
               mesh=mesh)
    def gather_kernel(tab_hbm, i_hbm, o_hbm):
        def body(i_vmem, o_vmem):
            pltpu.sync_copy(tab_hbm.at[i_vmem.at[0]], o_vmem)

        pltpu.emit_pipeline(
            body,
            grid=(n // GATHER_WINDOW,),
            in_specs=[pl.BlockSpec((1, GATHER_WINDOW),
                                   index_map=lambda i: (0, i))],
            out_specs=[pl.BlockSpec((GATHER_WINDOW, d_half),
                                    index_map=lambda i: (i, 0))],
            core_axis_name=("c", "s"),
            dimension_semantics=(pltpu.PARALLEL,),
        )(i_hbm, o_hbm)

    return gather_kernel(pairsum_i32, idx)


def _tc_body(mp_ref, emb_ref, w_ref, b_ref, out_ref):
    rows = BB * SQUARES
    mp = mp_ref[...].reshape(rows, SQUARES)
    acc = jnp.dot(mp, w_ref[...], preferred_element_type=jnp.float32)
    acc += emb_ref[...].astype(jnp.float32)
    acc += b_ref[...]
    out_ref[...] = acc.reshape(BB, SQUARES, D_MODEL)


def _tc_call(move_potentials, emb, W, b2):
    batch = move_potentials.shape[0]
    grid = batch // BB
    return pl.pallas_call(
        _tc_body,
        grid=(grid,),
        in_specs=[
            pl.BlockSpec((BB, SQUARES, SQUARES), lambda i: (i, 0, 0)),
            pl.BlockSpec((BB * SQUARES, D_MODEL), lambda i: (i, 0)),
            pl.BlockSpec((SQUARES, D_MODEL), lambda i: (0, 0)),
            pl.BlockSpec((1, D_MODEL), lambda i: (0, 0)),
        ],
        out_specs=pl.BlockSpec((BB, SQUARES, D_MODEL), lambda i: (i, 0, 0)),
        out_shape=jax.ShapeDtypeStruct((batch, SQUARES, D_MODEL), jnp.float32),
    )(move_potentials, emb, W, b2)


@jax.jit
def kernel(pieces, positions, move_potentials, piece_table, pos_table, W, b):
    batch = pieces.shape[0]
    n = batch * SQUARES
    pieces = pieces.astype(jnp.int32)
    positions = positions.astype(jnp.int32)
    pairsum = (piece_table[:, None, :] + pos_table[None, :, :]
               ).reshape(13 * SQUARES, D_MODEL).astype(jnp.bfloat16)
    pairsum_i32 = jax.lax.bitcast_convert_type(
        pairsum.reshape(13 * SQUARES, D_MODEL // 2, 2), jnp.int32)
    idx = (pieces * SQUARES + positions).reshape(1, n)
    b2 = b.reshape(1, D_MODEL)

    emb_i32 = _sc_gather(pairsum_i32, idx)
    emb = jax.lax.bitcast_convert_type(emb_i32, jnp.bfloat16).reshape(
        n, D_MODEL)
    return _tc_call(move_potentials, emb, W, b2)


# speedup vs baseline: 7.1021x; 7.1021x over previous
import jax
import jax.numpy as jnp
from jax.experimental import pallas as pl

D_MODEL = 512
SQUARES = 64
BB = 32


def _fused_body(pieces_ref, positions_ref, mp_ref, ptab_ref, qtab_ref,
                w_ref, b_ref, out_ref):
    rows = BB * SQUARES
    mp = mp_ref[...].reshape(rows, SQUARES)
    acc = jnp.dot(mp, w_ref[...], preferred_element_type=jnp.float32)

    p = pieces_ref[...][:, :, None]
    oh_p = (p == jax.lax.broadcasted_iota(jnp.int32, (BB, SQUARES, 16), 2)
            ).astype(jnp.float32).reshape(rows, 16)
    acc += jnp.dot(oh_p, ptab_ref[...], preferred_element_type=jnp.float32)

    q = positions_ref[...][:, :, None]
    oh_q = (q == jax.lax.broadcasted_iota(jnp.int32, (BB, SQUARES, SQUARES), 2)
            ).astype(jnp.float32).reshape(rows, SQUARES)
    acc += jnp.dot(oh_q, qtab_ref[...], preferred_element_type=jnp.float32)

    acc += b_ref[...]
    out_ref[...] = acc.reshape(BB, SQUARES, D_MODEL)


def _fused_call(pieces, positions, move_potentials, ptab, qtab, W, b2):
    batch = pieces.shape[0]
    grid = batch // BB
    return pl.pallas_call(
        _fused_body,
        grid=(grid,),
        in_specs=[
            pl.BlockSpec((BB, SQUARES), lambda i: (i, 0)),
            pl.BlockSpec((BB, SQUARES), lambda i: (i, 0)),
            pl.BlockSpec((BB, SQUARES, SQUARES), lambda i: (i, 0, 0)),
            pl.BlockSpec((16, D_MODEL), lambda i: (0, 0)),
            pl.BlockSpec((SQUARES, D_MODEL), lambda i: (0, 0)),
            pl.BlockSpec((SQUARES, D_MODEL), lambda i: (0, 0)),
            pl.BlockSpec((1, D_MODEL), lambda i: (0, 0)),
        ],
        out_specs=pl.BlockSpec((BB, SQUARES, D_MODEL), lambda i: (i, 0, 0)),
        out_shape=jax.ShapeDtypeStruct((batch, SQUARES, D_MODEL), jnp.float32),
    )(pieces, positions, move_potentials, ptab, qtab, W, b2)


@jax.jit
def kernel(pieces, positions, move_potentials, piece_table, pos_table, W, b):
    batch = pieces.shape[0]
    pieces = pieces.astype(jnp.int32)
    positions = positions.astype(jnp.int32)
    ptab = jnp.zeros((16, D_MODEL), jnp.float32).at[:13].set(piece_table)
    b2 = b.reshape(1, D_MODEL)
    return _fused_call(pieces, positions, move_potentials, ptab, pos_table,
                       W, b2)


# speedup vs baseline: 7.9726x; 1.1226x over previous
import jax
import jax.numpy as jnp
from jax.experimental import pallas as pl

D_MODEL = 512
SQUARES = 64
BB = 64


def _fused_body(pieces_ref, positions_ref, mp_ref, ptab_ref, qtab_ref,
                w_ref, b_ref, out_ref):
    rows = BB * SQUARES
    mp = mp_ref[...].reshape(rows, SQUARES)
    acc = jnp.dot(mp, w_ref[...], preferred_element_type=jnp.float32)

    p = pieces_ref[...][:, :, None]
    oh_p = (p == jax.lax.broadcasted_iota(jnp.int32, (BB, SQUARES, 13), 2)
            ).astype(jnp.float32).reshape(rows, 13)
    acc += jnp.dot(oh_p, ptab_ref[...], preferred_element_type=jnp.float32)

    q = positions_ref[...][:, :, None]
    oh_q = (q == jax.lax.broadcasted_iota(jnp.int32, (BB, SQUARES, SQUARES), 2)
            ).astype(jnp.float32).reshape(rows, SQUARES)
    acc += jnp.dot(oh_q, qtab_ref[...], preferred_element_type=jnp.float32)

    acc += b_ref[...]
    out_ref[...] = acc.reshape(BB, SQUARES, D_MODEL)


def _fused_call(pieces, positions, move_potentials, ptab, qtab, W, b2):
    batch = pieces.shape[0]
    grid = batch // BB
    return pl.pallas_call(
        _fused_body,
        grid=(grid,),
        in_specs=[
            pl.BlockSpec((BB, SQUARES), lambda i: (i, 0)),
            pl.BlockSpec((BB, SQUARES), lambda i: (i, 0)),
            pl.BlockSpec((BB, SQUARES, SQUARES), lambda i: (i, 0, 0)),
            pl.BlockSpec((13, D_MODEL), lambda i: (0, 0)),
            pl.BlockSpec((SQUARES, D_MODEL), lambda i: (0, 0)),
            pl.BlockSpec((SQUARES, D_MODEL), lambda i: (0, 0)),
            pl.BlockSpec((1, D_MODEL), lambda i: (0, 0)),
        ],
        out_specs=pl.BlockSpec((BB, SQUARES, D_MODEL), lambda i: (i, 0, 0)),
        out_shape=jax.ShapeDtypeStruct((batch, SQUARES, D_MODEL), jnp.float32),
    )(pieces, positions, move_potentials, ptab, qtab, W, b2)


@jax.jit
def kernel(pieces, positions, move_potentials, piece_table, pos_table, W, b):
    batch = pieces.shape[0]
    pieces = pieces.astype(jnp.int32)
    positions = positions.astype(jnp.int32)
    b2 = b.reshape(1, D_MODEL)
    return _fused_call(pieces, positions, move_potentials, piece_table,
                       pos_table, W, b2)
